# baseline (speedup 1.0000x reference)
import functools

import jax
import jax.numpy as jnp
from jax import lax
from jax.experimental import pallas as pl
from jax.experimental.pallas import tpu as pltpu
from jax.experimental.pallas import tpu_sc as plsc

N = 1024
D = 1024
NIDX = 21504
NW = 32
PER_W = NIDX // NW
K = 48
CHUNKS = PER_W // K

_mesh = plsc.VectorSubcoreMesh(core_axis_name="c", subcore_axis_name="s")


@functools.partial(
    pl.kernel,
    mesh=_mesh,
    out_type=jax.ShapeDtypeStruct((NIDX, D), jnp.float32),
    scratch_types=[
        pltpu.VMEM((PER_W,), jnp.int32),
        pltpu.VMEM((K, D), jnp.float32),
        pltpu.VMEM((K, D), jnp.float32),
        pltpu.SemaphoreType.DMA,
        pltpu.SemaphoreType.DMA,
    ],
)
def _gather_bench(table_hbm, idx_hbm, out_hbm, idx_v, rows0, rows1,
                  sem0, sem1):
    wid = lax.axis_index("s") * 2 + lax.axis_index("c")
    base = wid * PER_W
    pltpu.sync_copy(idx_hbm.at[pl.ds(base, PER_W)], idx_v)
    rows = [rows0, rows1]
    sems = [sem0, sem1]
    cps = [None, None]
    cps[0] = pltpu.async_copy(
        table_hbm.at[idx_v.at[pl.ds(0, K)]], rows[0], sems[0])
    for c in range(1, CHUNKS + 1):
        if c < CHUNKS:
            cps[c % 2] = pltpu.async_copy(
                table_hbm.at[idx_v.at[pl.ds(c * K, K)]], rows[c % 2],
                sems[c % 2])
        p = (c - 1) % 2
        cps[p].wait()
        pltpu.sync_copy(rows[p], out_hbm.at[pl.ds(base + (c - 1) * K, K)])


def kernel(inputs, hx, adj, W_gate, b_gate, W_c, b_c):
    table = inputs.reshape(N, D)
    idx = ((jnp.arange(NIDX, dtype=jnp.int32) * 577) % N).astype(jnp.int32)
    out = _gather_bench(table, idx)
    return jnp.broadcast_to(out[0:1, 0:1], (16, 65536))


# speedup vs baseline: 1.2496x; 1.2496x over previous
import jax
import jax.numpy as jnp
from jax.experimental import pallas as pl
from jax.experimental.pallas import tpu as pltpu

N = 1024
F = 64
B = 16
BB = 8
M = 5
W_BB = BB * F


def _dotT(a, b):
    return jax.lax.dot_general(
        a, b, (((0,), (0,)), ((), ())), preferred_element_type=jnp.float32)


def _dot(a, b):
    return jax.lax.dot_general(
        a, b, (((1,), (0,)), ((), ())), preferred_element_type=jnp.float32)


def _supports_kernel(adj_ref, s1_ref, s2_ref):
    a = adj_ref[...]
    d = jnp.sum(a, axis=1, keepdims=True)
    dinv = jnp.where(d > 0.0, 1.0 / d, 0.0)
    s1_ref[...] = (dinv * a).astype(jnp.bfloat16)
    d2 = jnp.sum(a, axis=0, keepdims=True)
    d2inv = jnp.where(d2 > 0.0, 1.0 / d2, 0.0)
    s2_ref[...] = (a * d2inv).astype(jnp.bfloat16)


def _cell_kernel(inp_ref, hx_ref, s1_ref, s2_ref, wg_ref, bg_ref, wc_ref,
                 bc_ref, out_ref):
    bf = jnp.bfloat16
    rw = s1_ref[...]
    s2 = s2_ref[...]

    x0 = jnp.concatenate(
        [inp_ref[b].astype(bf) for b in range(BB)]
        + [hx_ref[b].astype(bf) for b in range(BB)], axis=1)
    inp2b = x0[:, :W_BB]
    st2b = x0[:, W_BB:]

    y1 = _dotT(rw, x0).astype(bf)
    y2 = (2.0 * _dotT(rw, y1) - x0).astype(bf)
    z1 = _dot(s2, x0).astype(bf)
    z2 = (2.0 * _dot(s2, z1) - x0).astype(bf)

    wg = wg_ref[...]
    bg = bg_ref[...]
    wc = wc_ref[...]
    bc = bc_ref[...]

    st2p_parts = []
    u_parts = []
    for b in range(BB):
        lo, hi = b * F, (b + 1) * F
        slo, shi = W_BB + lo, W_BB + hi
        xb = jnp.concatenate(
            [inp2b[:, lo:hi], st2b[:, lo:hi], y1[:, lo:hi], y1[:, slo:shi],
             y2[:, lo:hi], y2[:, slo:shi], z1[:, lo:hi], z1[:, slo:shi],
             z2[:, lo:hi], z2[:, slo:shi]], axis=1)
        val = jax.nn.sigmoid(_dot(xb, wg) + bg)
        u_parts.append(val[:, F:].astype(bf))
        st2p_parts.append((val[:, :F] * hx_ref[b]).astype(bf))

    st2pb = jnp.concatenate(st2p_parts, axis=1)
    r1 = _dotT(rw, st2pb).astype(bf)
    r2 = (2.0 * _dotT(rw, r1) - st2pb).astype(bf)
    r3 = _dot(s2, st2pb).astype(bf)
    r4 = (2.0 * _dot(s2, r3) - st2pb).astype(bf)

    for b in range(BB):
        lo, hi = b * F, (b + 1) * F
        xb = jnp.concatenate(
            [inp2b[:, lo:hi], st2pb[:, lo:hi], y1[:, lo:hi], r1[:, lo:hi],
             y2[:, lo:hi], r2[:, lo:hi], z1[:, lo:hi], r3[:, lo:hi],
             z2[:, lo:hi], r4[:, lo:hi]], axis=1)
        c = jnp.tanh(_dot(xb, wc) + bc)
        u = u_parts[b].astype(jnp.float32)
        out_ref[b] = u * hx_ref[b] + (1.0 - u) * c


def kernel(inputs, hx, adj, W_gate, b_gate, W_c, b_c):
    inp3 = inputs.reshape(B, N, F)
    hx3 = hx.reshape(B, N, F)
    wg = W_gate.reshape(2 * F, M, 2 * F).transpose(1, 0, 2).reshape(
        M * 2 * F, 2 * F).astype(jnp.bfloat16)
    wc = W_c.reshape(2 * F, M, F).transpose(1, 0, 2).reshape(
        M * 2 * F, F).astype(jnp.bfloat16)
    bg = b_gate.reshape(1, 2 * F)
    bc = b_c.reshape(1, F)

    s1, s2 = pl.pallas_call(
        _supports_kernel,
        out_shape=[
            jax.ShapeDtypeStruct((N, N), jnp.bfloat16),
            jax.ShapeDtypeStruct((N, N), jnp.bfloat16),
        ],
    )(adj)

    out = pl.pallas_call(
        _cell_kernel,
        grid=(B // BB,),
        in_specs=[
            pl.BlockSpec((BB, N, F), lambda i: (i, 0, 0)),
            pl.BlockSpec((BB, N, F), lambda i: (i, 0, 0)),
            pl.BlockSpec((N, N), lambda i: (0, 0)),
            pl.BlockSpec((N, N), lambda i: (0, 0)),
            pl.BlockSpec((M * 2 * F, 2 * F), lambda i: (0, 0)),
            pl.BlockSpec((1, 2 * F), lambda i: (0, 0)),
            pl.BlockSpec((M * 2 * F, F), lambda i: (0, 0)),
            pl.BlockSpec((1, F), lambda i: (0, 0)),
        ],
        out_specs=pl.BlockSpec((BB, N, F), lambda i: (i, 0, 0)),
        out_shape=jax.ShapeDtypeStruct((B, N, F), jnp.float32),
    )(inp3, hx3, s1, s2, wg, bg, wc, bc)
    return out.reshape(B, N * F)
